# baseline (speedup 1.0000x reference)
import functools

import jax
import jax.numpy as jnp
from jax import lax
from jax.experimental import pallas as pl
from jax.experimental.pallas import tpu as pltpu
from jax.experimental.pallas import tpu_sc as plsc

_N = 100000
_NUM_SEG = 100
_W = 0.0001
_NW = 32
_CHUNK = 3136
_SEGP = 128
_ITERS = _CHUNK // 16


def _sc_body(pred_hbm, vel_hbm, m_hbm, b_hbm, out_hbm,
             pbuf, vbuf, mv, bv, a0, a1, a2, r0, r1, r2, sem):
    wid = lax.axis_index("s") * 2 + lax.axis_index("c")
    lstart = wid * _CHUNK
    base = jnp.minimum(lstart, _N - _CHUNK)

    cps = [
        pltpu.async_copy(pred_hbm.at[pl.ds(base * 6, _CHUNK * 6)], pbuf, sem),
        pltpu.async_copy(vel_hbm.at[pl.ds(base * 3, _CHUNK * 3)], vbuf, sem),
        pltpu.async_copy(m_hbm.at[pl.ds(base, _CHUNK)], mv, sem),
        pltpu.async_copy(b_hbm.at[pl.ds(base, _CHUNK)], bv, sem),
    ]

    zeros = jnp.zeros((16,), jnp.float32)

    def zero_body(i, carry):
        a0[pl.ds(i * 16, 16)] = zeros
        a1[pl.ds(i * 16, 16)] = zeros
        a2[pl.ds(i * 16, 16)] = zeros
        return carry

    lax.fori_loop(0, 16 * _SEGP // 16, zero_body, 0)

    for cp in cps:
        cp.wait()

    iota = lax.broadcasted_iota(jnp.int32, (16,), 0)
    i6 = iota * 6
    i3 = iota * 3
    lane_off = iota * _SEGP

    def it(i, carry):
        s = i * 16
        s6 = s * 6
        s3 = s * 3
        idx = bv[pl.ds(s, 16)] + lane_off
        m = mv[pl.ds(s, 16)]
        for c, a in ((0, a0), (1, a1), (2, a2)):
            pv_c = plsc.load_gather(pbuf, [i6 + (s6 + 3 + c)])
            v_c = plsc.load_gather(vbuf, [i3 + (s3 + c)])
            plsc.addupdate_scatter(a, [idx], m * (pv_c - v_c))
        return carry

    lax.fori_loop((lstart - base) // 16, _ITERS, it, 0)

    for a, r in ((a0, r0), (a1, r1), (a2, r2)):
        for k in range(_SEGP // 16):
            tot = a[pl.ds(k * 16, 16)]
            for lane in range(1, 16):
                tot = tot + a[pl.ds(lane * _SEGP + k * 16, 16)]
            r[pl.ds(k * 16, 16)] = tot

    obase = wid * 3 * _SEGP
    pltpu.sync_copy(r0, out_hbm.at[pl.ds(obase + 0 * _SEGP, _SEGP)])
    pltpu.sync_copy(r1, out_hbm.at[pl.ds(obase + 1 * _SEGP, _SEGP)])
    pltpu.sync_copy(r2, out_hbm.at[pl.ds(obase + 2 * _SEGP, _SEGP)])


_sc_partials = functools.partial(
    pl.kernel,
    mesh=plsc.VectorSubcoreMesh(core_axis_name="c", subcore_axis_name="s"),
    out_type=jax.ShapeDtypeStruct((_NW * 3 * _SEGP,), jnp.float32),
    compiler_params=pltpu.CompilerParams(needs_layout_passes=False),
    scratch_types=[
        pltpu.VMEM((_CHUNK * 6,), jnp.float32),
        pltpu.VMEM((_CHUNK * 3,), jnp.float32),
        pltpu.VMEM((_CHUNK,), jnp.float32),
        pltpu.VMEM((_CHUNK,), jnp.int32),
        pltpu.VMEM((16 * _SEGP,), jnp.float32),
        pltpu.VMEM((16 * _SEGP,), jnp.float32),
        pltpu.VMEM((16 * _SEGP,), jnp.float32),
        pltpu.VMEM((_SEGP,), jnp.float32),
        pltpu.VMEM((_SEGP,), jnp.float32),
        pltpu.VMEM((_SEGP,), jnp.float32),
        pltpu.SemaphoreType.DMA,
    ],
)(_sc_body)


def _tc_body(p_ref, o_ref):
    x = p_ref[...]
    s = jnp.sum(x, axis=0)
    o_ref[0, 0] = jnp.sum(s * s) * (_W / (3.0 * _NUM_SEG))


_tc_finish = pl.pallas_call(
    _tc_body,
    out_shape=jax.ShapeDtypeStruct((1, 1), jnp.float32),
    out_specs=pl.BlockSpec(memory_space=pltpu.SMEM),
)


def kernel(pred, vel, y, mass, batch):
    del y
    partials = _sc_partials(pred.reshape(-1), vel.reshape(-1), mass,
                            batch.astype(jnp.int32))
    return _tc_finish(partials.reshape(_NW, 3, _SEGP))[0, 0]


# speedup vs baseline: 4.2626x; 4.2626x over previous
import functools

import jax
import jax.numpy as jnp
from jax import lax
from jax.experimental import pallas as pl
from jax.experimental.pallas import tpu as pltpu
from jax.experimental.pallas import tpu_sc as plsc

_N = 100000
_NUM_SEG = 100
_W = 0.0001
_NW = 32
_CHUNK = 3136
_SEGP = 128
_ITERS = _CHUNK // 16


def _sc_body(pv_hbm, v_hbm, m_hbm, b_hbm, out_hbm,
             p0, p1, p2, v0, v1, v2, mv, bv, a0, a1, a2, r0, r1, r2, sem):
    wid = lax.axis_index("s") * 2 + lax.axis_index("c")
    lstart = wid * _CHUNK
    base = jnp.minimum(lstart, _N - _CHUNK)

    cps = [
        pltpu.async_copy(pv_hbm.at[pl.ds(0 * _N + base, _CHUNK)], p0, sem),
        pltpu.async_copy(pv_hbm.at[pl.ds(1 * _N + base, _CHUNK)], p1, sem),
        pltpu.async_copy(pv_hbm.at[pl.ds(2 * _N + base, _CHUNK)], p2, sem),
        pltpu.async_copy(v_hbm.at[pl.ds(0 * _N + base, _CHUNK)], v0, sem),
        pltpu.async_copy(v_hbm.at[pl.ds(1 * _N + base, _CHUNK)], v1, sem),
        pltpu.async_copy(v_hbm.at[pl.ds(2 * _N + base, _CHUNK)], v2, sem),
        pltpu.async_copy(m_hbm.at[pl.ds(base, _CHUNK)], mv, sem),
        pltpu.async_copy(b_hbm.at[pl.ds(base, _CHUNK)], bv, sem),
    ]

    zeros = jnp.zeros((16,), jnp.float32)

    def zero_body(i, carry):
        a0[pl.ds(i * 16, 16)] = zeros
        a1[pl.ds(i * 16, 16)] = zeros
        a2[pl.ds(i * 16, 16)] = zeros
        return carry

    lax.fori_loop(0, 16 * _SEGP // 16, zero_body, 0)

    for cp in cps:
        cp.wait()

    lane_off = lax.broadcasted_iota(jnp.int32, (16,), 0) * _SEGP

    def it(i, carry):
        s = i * 16
        idx = bv[pl.ds(s, 16)] + lane_off
        m = mv[pl.ds(s, 16)]
        plsc.addupdate_scatter(a0, [idx], m * (p0[pl.ds(s, 16)] - v0[pl.ds(s, 16)]))
        plsc.addupdate_scatter(a1, [idx], m * (p1[pl.ds(s, 16)] - v1[pl.ds(s, 16)]))
        plsc.addupdate_scatter(a2, [idx], m * (p2[pl.ds(s, 16)] - v2[pl.ds(s, 16)]))
        return carry

    lax.fori_loop((lstart - base) // 16, _ITERS, it, 0)

    for a, r in ((a0, r0), (a1, r1), (a2, r2)):
        for k in range(_SEGP // 16):
            tot = a[pl.ds(k * 16, 16)]
            for lane in range(1, 16):
                tot = tot + a[pl.ds(lane * _SEGP + k * 16, 16)]
            r[pl.ds(k * 16, 16)] = tot

    obase = wid * 3 * _SEGP
    pltpu.sync_copy(r0, out_hbm.at[pl.ds(obase + 0 * _SEGP, _SEGP)])
    pltpu.sync_copy(r1, out_hbm.at[pl.ds(obase + 1 * _SEGP, _SEGP)])
    pltpu.sync_copy(r2, out_hbm.at[pl.ds(obase + 2 * _SEGP, _SEGP)])


_sc_partials = functools.partial(
    pl.kernel,
    mesh=plsc.VectorSubcoreMesh(core_axis_name="c", subcore_axis_name="s"),
    out_type=jax.ShapeDtypeStruct((_NW * 3 * _SEGP,), jnp.float32),
    compiler_params=pltpu.CompilerParams(needs_layout_passes=False),
    scratch_types=[
        pltpu.VMEM((_CHUNK,), jnp.float32),
        pltpu.VMEM((_CHUNK,), jnp.float32),
        pltpu.VMEM((_CHUNK,), jnp.float32),
        pltpu.VMEM((_CHUNK,), jnp.float32),
        pltpu.VMEM((_CHUNK,), jnp.float32),
        pltpu.VMEM((_CHUNK,), jnp.float32),
        pltpu.VMEM((_CHUNK,), jnp.float32),
        pltpu.VMEM((_CHUNK,), jnp.int32),
        pltpu.VMEM((16 * _SEGP,), jnp.float32),
        pltpu.VMEM((16 * _SEGP,), jnp.float32),
        pltpu.VMEM((16 * _SEGP,), jnp.float32),
        pltpu.VMEM((_SEGP,), jnp.float32),
        pltpu.VMEM((_SEGP,), jnp.float32),
        pltpu.VMEM((_SEGP,), jnp.float32),
        pltpu.SemaphoreType.DMA,
    ],
)(_sc_body)


def _tc_body(p_ref, o_ref):
    x = p_ref[...]
    s = jnp.sum(x, axis=0)
    o_ref[0, 0] = jnp.sum(s * s) * (_W / (3.0 * _NUM_SEG))


_tc_finish = pl.pallas_call(
    _tc_body,
    out_shape=jax.ShapeDtypeStruct((1, 1), jnp.float32),
    out_specs=pl.BlockSpec(memory_space=pltpu.SMEM),
)


def kernel(pred, vel, y, mass, batch):
    del y
    pvT = pred[:, 3:6].T.reshape(-1)
    vT = vel.T.reshape(-1)
    partials = _sc_partials(pvT, vT, mass, batch.astype(jnp.int32))
    return _tc_finish(partials.reshape(_NW, 3, _SEGP))[0, 0]


# speedup vs baseline: 5.7486x; 1.3486x over previous
import functools

import jax
import jax.numpy as jnp
from jax import lax
from jax.experimental import pallas as pl
from jax.experimental.pallas import tpu as pltpu
from jax.experimental.pallas import tpu_sc as plsc

_N = 100000
_NUM_SEG = 100
_W = 0.0001
_NW = 32
_CHUNK = 3136
_SEGP = 128
_LSTRIDE = 129
_ITERS = _CHUNK // 16


def _sc_body(pv_hbm, v_hbm, m_hbm, b_hbm, out_hbm,
             p0, p1, p2, v0, v1, v2, mv, bv, a0, a1, a2, r0, r1, r2, sem):
    wid = lax.axis_index("s") * 2 + lax.axis_index("c")
    lstart = wid * _CHUNK
    base = jnp.minimum(lstart, _N - _CHUNK)

    cps = [
        pltpu.async_copy(pv_hbm.at[pl.ds(0 * _N + base, _CHUNK)], p0, sem),
        pltpu.async_copy(pv_hbm.at[pl.ds(1 * _N + base, _CHUNK)], p1, sem),
        pltpu.async_copy(pv_hbm.at[pl.ds(2 * _N + base, _CHUNK)], p2, sem),
        pltpu.async_copy(v_hbm.at[pl.ds(0 * _N + base, _CHUNK)], v0, sem),
        pltpu.async_copy(v_hbm.at[pl.ds(1 * _N + base, _CHUNK)], v1, sem),
        pltpu.async_copy(v_hbm.at[pl.ds(2 * _N + base, _CHUNK)], v2, sem),
        pltpu.async_copy(m_hbm.at[pl.ds(base, _CHUNK)], mv, sem),
        pltpu.async_copy(b_hbm.at[pl.ds(base, _CHUNK)], bv, sem),
    ]

    zeros = jnp.zeros((16,), jnp.float32)

    def zero_body(i, carry):
        a0[pl.ds(i * 16, 16)] = zeros
        a1[pl.ds(i * 16, 16)] = zeros
        a2[pl.ds(i * 16, 16)] = zeros
        return carry

    lax.fori_loop(0, 16 * _LSTRIDE // 16, zero_body, 0)

    lane_off = lax.broadcasted_iota(jnp.int32, (16,), 0) * _LSTRIDE

    def body(s):
        idx = bv[pl.ds(s, 16)] + lane_off
        m = mv[pl.ds(s, 16)]
        plsc.addupdate_scatter(a0, [idx], m * (p0[pl.ds(s, 16)] - v0[pl.ds(s, 16)]))
        plsc.addupdate_scatter(a1, [idx], m * (p1[pl.ds(s, 16)] - v1[pl.ds(s, 16)]))
        plsc.addupdate_scatter(a2, [idx], m * (p2[pl.ds(s, 16)] - v2[pl.ds(s, 16)]))

    for cp in cps:
        cp.wait()
    plsc.parallel_loop((lstart - base) // 16 * 16, _CHUNK, step=16, unroll=4)(body)

    def red_body(k, carry):
        o = k * 16
        for a, r in ((a0, r0), (a1, r1), (a2, r2)):
            tot = a[pl.ds(o, 16)]
            for lane in range(1, 16):
                tot = tot + a[pl.ds(lane * _LSTRIDE + o, 16)]
            r[pl.ds(o, 16)] = tot
        return carry

    lax.fori_loop(0, _SEGP // 16, red_body, 0)

    pltpu.sync_copy(r0, out_hbm.at[pl.ds((0 * _NW + wid) * _SEGP, _SEGP)])
    pltpu.sync_copy(r1, out_hbm.at[pl.ds((1 * _NW + wid) * _SEGP, _SEGP)])
    pltpu.sync_copy(r2, out_hbm.at[pl.ds((2 * _NW + wid) * _SEGP, _SEGP)])


_sc_partials = functools.partial(
    pl.kernel,
    mesh=plsc.VectorSubcoreMesh(core_axis_name="c", subcore_axis_name="s"),
    out_type=jax.ShapeDtypeStruct((_NW * 3 * _SEGP,), jnp.float32),
    compiler_params=pltpu.CompilerParams(needs_layout_passes=False),
    scratch_types=[
        pltpu.VMEM((_CHUNK,), jnp.float32),
        pltpu.VMEM((_CHUNK,), jnp.float32),
        pltpu.VMEM((_CHUNK,), jnp.float32),
        pltpu.VMEM((_CHUNK,), jnp.float32),
        pltpu.VMEM((_CHUNK,), jnp.float32),
        pltpu.VMEM((_CHUNK,), jnp.float32),
        pltpu.VMEM((_CHUNK,), jnp.float32),
        pltpu.VMEM((_CHUNK,), jnp.int32),
        pltpu.VMEM((16 * _LSTRIDE,), jnp.float32),
        pltpu.VMEM((16 * _LSTRIDE,), jnp.float32),
        pltpu.VMEM((16 * _LSTRIDE,), jnp.float32),
        pltpu.VMEM((_SEGP,), jnp.float32),
        pltpu.VMEM((_SEGP,), jnp.float32),
        pltpu.VMEM((_SEGP,), jnp.float32),
        pltpu.SemaphoreType.DMA,
    ],
)(_sc_body)


def _tc_body(p_ref, o_ref):
    x = p_ref[...].reshape(3 * _NW, _SEGP)
    acc = jnp.float32(0.0)
    for c in range(3):
        s = jnp.sum(x[c * _NW:(c + 1) * _NW, :], axis=0)
        acc = acc + jnp.sum(s * s)
    o_ref[0, 0] = acc * (_W / (3.0 * _NUM_SEG))


_tc_finish = pl.pallas_call(
    _tc_body,
    out_shape=jax.ShapeDtypeStruct((1, 1), jnp.float32),
    out_specs=pl.BlockSpec(memory_space=pltpu.SMEM),
)


def kernel(pred, vel, y, mass, batch):
    del y
    pvT = lax.reshape(pred[:, 3:6], (3 * _N,), dimensions=(1, 0))
    vT = lax.reshape(vel, (3 * _N,), dimensions=(1, 0))
    partials = _sc_partials(pvT, vT, mass, batch.astype(jnp.int32))
    return _tc_finish(partials)[0, 0]


# speedup vs baseline: 5.7869x; 1.0067x over previous
import functools

import jax
import jax.numpy as jnp
from jax import lax
from jax.experimental import pallas as pl
from jax.experimental.pallas import tpu as pltpu
from jax.experimental.pallas import tpu_sc as plsc

_N = 100000
_NUM_SEG = 100
_W = 0.0001
_NW = 32
_CHUNK = 3136
_SEGP = 128
_LSTRIDE = 129
_ITERS = _CHUNK // 16


def _sc_body(pv_hbm, v_hbm, m_hbm, b_hbm, out_hbm,
             p0, p1, p2, v0, v1, v2, mv, bv, a0, a1, a2, r0, r1, r2, sem):
    wid = lax.axis_index("s") * 2 + lax.axis_index("c")
    lstart = wid * _CHUNK
    base = jnp.minimum(lstart, _N - _CHUNK)

    cps = [
        pltpu.async_copy(pv_hbm.at[pl.ds(0 * _N + base, _CHUNK)], p0, sem),
        pltpu.async_copy(pv_hbm.at[pl.ds(1 * _N + base, _CHUNK)], p1, sem),
        pltpu.async_copy(pv_hbm.at[pl.ds(2 * _N + base, _CHUNK)], p2, sem),
        pltpu.async_copy(v_hbm.at[pl.ds(0 * _N + base, _CHUNK)], v0, sem),
        pltpu.async_copy(v_hbm.at[pl.ds(1 * _N + base, _CHUNK)], v1, sem),
        pltpu.async_copy(v_hbm.at[pl.ds(2 * _N + base, _CHUNK)], v2, sem),
        pltpu.async_copy(m_hbm.at[pl.ds(base, _CHUNK)], mv, sem),
        pltpu.async_copy(b_hbm.at[pl.ds(base, _CHUNK)], bv, sem),
    ]

    zeros = jnp.zeros((16,), jnp.float32)

    def zero_body(i, carry):
        a0[pl.ds(i * 16, 16)] = zeros
        a1[pl.ds(i * 16, 16)] = zeros
        a2[pl.ds(i * 16, 16)] = zeros
        return carry

    lax.fori_loop(0, 16 * _LSTRIDE // 16, zero_body, 0)

    lane_off = lax.broadcasted_iota(jnp.int32, (16,), 0) * _LSTRIDE

    def body(s):
        idx = bv[pl.ds(s, 16)] + lane_off
        m = mv[pl.ds(s, 16)]
        plsc.addupdate_scatter(a0, [idx], m * (p0[pl.ds(s, 16)] - v0[pl.ds(s, 16)]))
        plsc.addupdate_scatter(a1, [idx], m * (p1[pl.ds(s, 16)] - v1[pl.ds(s, 16)]))
        plsc.addupdate_scatter(a2, [idx], m * (p2[pl.ds(s, 16)] - v2[pl.ds(s, 16)]))

    for cp in cps:
        cp.wait()
    plsc.parallel_loop((lstart - base) // 16 * 16, _CHUNK, step=16, unroll=2)(body)

    def red_body(k, carry):
        o = k * 16
        for a, r in ((a0, r0), (a1, r1), (a2, r2)):
            tot = a[pl.ds(o, 16)]
            for lane in range(1, 16):
                tot = tot + a[pl.ds(lane * _LSTRIDE + o, 16)]
            r[pl.ds(o, 16)] = tot
        return carry

    lax.fori_loop(0, _SEGP // 16, red_body, 0)

    pltpu.sync_copy(r0, out_hbm.at[pl.ds((0 * _NW + wid) * _SEGP, _SEGP)])
    pltpu.sync_copy(r1, out_hbm.at[pl.ds((1 * _NW + wid) * _SEGP, _SEGP)])
    pltpu.sync_copy(r2, out_hbm.at[pl.ds((2 * _NW + wid) * _SEGP, _SEGP)])


_sc_partials = functools.partial(
    pl.kernel,
    mesh=plsc.VectorSubcoreMesh(core_axis_name="c", subcore_axis_name="s"),
    out_type=jax.ShapeDtypeStruct((_NW * 3 * _SEGP,), jnp.float32),
    compiler_params=pltpu.CompilerParams(needs_layout_passes=False),
    scratch_types=[
        pltpu.VMEM((_CHUNK,), jnp.float32),
        pltpu.VMEM((_CHUNK,), jnp.float32),
        pltpu.VMEM((_CHUNK,), jnp.float32),
        pltpu.VMEM((_CHUNK,), jnp.float32),
        pltpu.VMEM((_CHUNK,), jnp.float32),
        pltpu.VMEM((_CHUNK,), jnp.float32),
        pltpu.VMEM((_CHUNK,), jnp.float32),
        pltpu.VMEM((_CHUNK,), jnp.int32),
        pltpu.VMEM((16 * _LSTRIDE,), jnp.float32),
        pltpu.VMEM((16 * _LSTRIDE,), jnp.float32),
        pltpu.VMEM((16 * _LSTRIDE,), jnp.float32),
        pltpu.VMEM((_SEGP,), jnp.float32),
        pltpu.VMEM((_SEGP,), jnp.float32),
        pltpu.VMEM((_SEGP,), jnp.float32),
        pltpu.SemaphoreType.DMA,
    ],
)(_sc_body)


def _tc_body(p_ref, o_ref):
    x = p_ref[...].reshape(3 * _NW, _SEGP)
    acc = jnp.float32(0.0)
    for c in range(3):
        s = jnp.sum(x[c * _NW:(c + 1) * _NW, :], axis=0)
        acc = acc + jnp.sum(s * s)
    o_ref[0, 0] = acc * (_W / (3.0 * _NUM_SEG))


_tc_finish = pl.pallas_call(
    _tc_body,
    out_shape=jax.ShapeDtypeStruct((1, 1), jnp.float32),
    out_specs=pl.BlockSpec(memory_space=pltpu.SMEM),
)


def kernel(pred, vel, y, mass, batch):
    del y
    pvT = lax.reshape(pred[:, 3:6], (3 * _N,), dimensions=(1, 0))
    vT = lax.reshape(vel, (3 * _N,), dimensions=(1, 0))
    partials = _sc_partials(pvT, vT, mass, batch.astype(jnp.int32))
    return _tc_finish(partials)[0, 0]
